# Initial kernel scaffold; baseline (speedup 1.0000x reference)
import jax
import jax.numpy as jnp
from jax.experimental import pallas as pl


def kernel(x, W):
    raise NotImplementedError("write your pallas kernel here")


# speedup vs baseline: 116.1369x; 116.1369x over previous
import functools

import jax
import jax.numpy as jnp
from jax import lax
from jax.experimental import pallas as pl
from jax.experimental.pallas import tpu as pltpu
from jax.experimental.pallas import tpu_sc as plsc

N = 32768
D = 768
E = 64
K = 8
CAP = 512
SLOTS = N * K
LBW = 0.01
ZW = 0.001

TILE = 512
NT = N // TILE

NCORE = 2
NSUB = 16
CH = SLOTS // NSUB
EPC = E // NCORE
EPT = EPC // NSUB
HB = 1024
SENT = 0x7FFFFFFF
SROW = 16


def _stage1_body(x_ref, w_ref, idx_ref, probs_ref, colsum_ref, zsum_ref):
    t = pl.program_id(0)
    x = x_ref[...]
    w = w_ref[...]
    logits = lax.dot_general(x, w, (((1,), (1,)), ((), ())),
                             preferred_element_type=jnp.float32)
    rowmax = jnp.max(logits, axis=-1, keepdims=True)
    ex = jnp.exp(logits - rowmax)
    sumex = jnp.sum(ex, axis=-1, keepdims=True)

    @pl.when(t == 0)
    def _():
        colsum_ref[...] = jnp.zeros_like(colsum_ref)
        zsum_ref[...] = jnp.zeros_like(zsum_ref)

    colsum_ref[...] += jnp.sum(ex / sumex, axis=0, keepdims=True)
    lse = jnp.log(sumex) + rowmax
    zsum_ref[...] += jnp.sum(lse * lse).reshape(1, 1)

    iota = lax.broadcasted_iota(jnp.int32, (TILE, E), 1)
    cur = logits
    vals = []
    idxs = []
    for _k in range(K):
        m = jnp.max(cur, axis=-1, keepdims=True)
        am = jnp.min(jnp.where(cur == m, iota, E), axis=-1, keepdims=True)
        vals.append(m)
        idxs.append(am)
        cur = jnp.where(iota == am, -jnp.inf, cur)
    v = jnp.concatenate(vals, axis=1)
    topi = jnp.concatenate(idxs, axis=1)
    ev = jnp.exp(v - v[:, :1])
    p = ev / jnp.sum(ev, axis=-1, keepdims=True)
    p = p / jnp.maximum(jnp.sum(p, axis=-1, keepdims=True), 1e-8)
    idx_ref[...] = topi
    probs_ref[...] = p


def _stage1(x, W):
    return pl.pallas_call(
        _stage1_body,
        grid=(NT,),
        in_specs=[
            pl.BlockSpec((TILE, D), lambda t: (t, 0)),
            pl.BlockSpec((E, D), lambda t: (0, 0)),
        ],
        out_specs=[
            pl.BlockSpec((TILE, K), lambda t: (t, 0)),
            pl.BlockSpec((TILE, K), lambda t: (t, 0)),
            pl.BlockSpec((1, E), lambda t: (0, 0)),
            pl.BlockSpec((1, 1), lambda t: (0, 0)),
        ],
        out_shape=[
            jax.ShapeDtypeStruct((N, K), jnp.int32),
            jax.ShapeDtypeStruct((N, K), jnp.float32),
            jax.ShapeDtypeStruct((1, E), jnp.float32),
            jax.ShapeDtypeStruct((1, 1), jnp.float32),
        ],
    )(x, W)


def _lane_field(row, k):
    lane = lax.iota(jnp.int32, 16)
    return jnp.sum(jnp.where(lane == k, row, 0))


def _sc_body(pbits_hbm, experts_hbm, out_hbm,
             pchunk, echunk, hist, macc, state, shist, sstate):
    c = lax.axis_index("c")
    wid = lax.axis_index("s")
    lane = lax.iota(jnp.int32, 16)
    zero16 = jnp.zeros((16,), jnp.int32)

    pltpu.sync_copy(pbits_hbm.at[pl.ds(wid * CH, CH)], pchunk)
    pltpu.sync_copy(experts_hbm.at[pl.ds(wid * CH, CH)], echunk)

    init_row = jnp.where(lane == 2, jnp.int32(CAP), 0)
    for e in range(EPC):
        state[pl.ds(e * SROW, 16)] = init_row

    e0 = wid * EPT
    off = e0 * HB

    for lvl in range(5):
        @pl.loop(0, EPC * HB, step=16)
        def _(i):
            hist[pl.ds(i, 16)] = zero16

        @pl.loop(0, CH, step=16)
        def _(i):
            pb = pchunk[pl.ds(i, 16)]
            ev = echunk[pl.ds(i, 16)]
            eloc = lax.bitwise_and(ev, EPC - 1)
            mycore = lax.shift_right_logical(ev, 5) == c
            tw = plsc.load_gather(state, [eloc * SROW])
            if lvl == 0:
                pend = mycore
                bucket = lax.shift_right_logical(pb, 20)
            elif lvl == 1:
                pend = mycore & (lax.shift_right_logical(pb, 20)
                                 == lax.shift_right_logical(tw, 20))
                bucket = lax.bitwise_and(lax.shift_right_logical(pb, 10), HB - 1)
            elif lvl == 2:
                pend = mycore & (lax.shift_right_logical(pb, 10)
                                 == lax.shift_right_logical(tw, 10))
                bucket = lax.bitwise_and(pb, HB - 1)
            elif lvl == 3:
                inv = (SLOTS - 1) - (wid * CH + i + lane)
                pend = mycore & (pb == tw)
                bucket = lax.shift_right_logical(inv, 8)
            else:
                tiw = plsc.load_gather(state, [eloc * SROW + 1])
                inv = (SLOTS - 1) - (wid * CH + i + lane)
                pend = mycore & (pb == tw) & (
                    lax.shift_right_logical(inv, 8)
                    == lax.shift_right_logical(tiw, 8))
                bucket = lax.bitwise_and(inv, 255)
            addr = eloc * HB + bucket
            cnt, last = plsc.scan_count(addr, mask=pend)
            plsc.addupdate_scatter(hist, [addr], cnt, mask=last)

        pltpu.sync_copy(hist, shist.at[wid])
        plsc.subcore_barrier()
        for t2 in range(NSUB):
            pltpu.sync_copy(shist.at[t2, pl.ds(off, EPT * HB)],
                            hist.at[pl.ds(t2 * EPT * HB, EPT * HB)])

        @pl.loop(0, EPT * HB, step=16)
        def _(j):
            acc = hist[pl.ds(j, 16)]
            for t2 in range(1, NSUB):
                acc = acc + hist[pl.ds(t2 * EPT * HB + j, 16)]
            macc[pl.ds(j, 16)] = acc

        for ex in range(EPT):
            eloc = e0 + ex
            row = state[pl.ds(eloc * SROW, 16)]
            tpbw = _lane_field(row, 0)
            tinvw = _lane_field(row, 1)
            r = _lane_field(row, 2)
            tpbf = _lane_field(row, 3)
            tinvf = _lane_field(row, 4)
            notdone = tpbw != SENT

            def scan_step(jj, carry, _ex=ex, _r=r):
                suffix, found, bstar, sexcl = carry
                j = 63 - jj
                v = macc[pl.ds(_ex * HB + j * 16, 16)]
                vrev = lax.rev(v, (0,))
                cum = plsc.cumsum(vrev)
                tot = jnp.sum(v)
                hit = (found == 0) & (suffix + tot >= _r)
                target = _r - suffix
                ffs = jnp.max(plsc.all_reduce_ffs(cum >= target))
                before = jnp.sum(jnp.where(lane < ffs, vrev, 0))
                cand_b = j * 16 + (15 - ffs)
                cand_se = suffix + before
                bstar = jnp.where(hit, cand_b, bstar)
                sexcl = jnp.where(hit, cand_se, sexcl)
                found = found | jnp.where(hit, 1, 0)
                return suffix + tot, found, bstar, sexcl

            z = jnp.int32(0)
            _, found, bstar, sexcl = lax.fori_loop(0, 64, scan_step,
                                                   (z, z, z, z))
            keepall = found == 0
            newr = r - sexcl
            if lvl == 0:
                ntpbw = lax.shift_left(bstar, 20)
                ntinvw = tinvw
            elif lvl == 1:
                ntpbw = tpbw | lax.shift_left(bstar, 10)
                ntinvw = tinvw
            elif lvl == 2:
                ntpbw = tpbw | bstar
                ntinvw = tinvw
            elif lvl == 3:
                ntpbw = tpbw
                ntinvw = lax.shift_left(bstar, 8)
            else:
                ntpbw = tpbw
                ntinvw = tinvw | bstar
            if lvl <= 2:
                ntpbf, ntinvf = ntpbw, tinvf
            else:
                ntpbf, ntinvf = tpbf, ntinvw
            live = notdone & jnp.logical_not(keepall)
            wtpbw = jnp.where(notdone,
                              jnp.where(keepall, jnp.int32(SENT), ntpbw), tpbw)
            wtinvw = jnp.where(live, ntinvw, tinvw)
            wr = jnp.where(live, newr, r)
            wtpbf = jnp.where(live, ntpbf, tpbf)
            wtinvf = jnp.where(live, ntinvf, tinvf)
            nrow = jnp.where(
                lane == 0, wtpbw,
                jnp.where(lane == 1, wtinvw,
                jnp.where(lane == 2, wr,
                jnp.where(lane == 3, wtpbf,
                jnp.where(lane == 4, wtinvf,
                jnp.where(lane == 5, lax.shift_right_logical(wtpbf, 15),
                jnp.where(lane == 6, lax.bitwise_and(wtpbf, 0x7FFF), 0)))))))
            state[pl.ds(eloc * SROW, 16)] = nrow

        pltpu.sync_copy(state.at[pl.ds(e0 * SROW, EPT * SROW)],
                        sstate.at[pl.ds(e0 * SROW, EPT * SROW)])
        plsc.subcore_barrier()
        pltpu.sync_copy(sstate, state)

    pltpu.sync_copy(state.at[pl.ds(e0 * SROW, EPT * SROW)],
                    out_hbm.at[pl.ds((c * EPC + e0) * SROW, EPT * SROW)])


def _sc_capacity(pbits_flat, experts_flat):
    mesh = plsc.VectorSubcoreMesh(core_axis_name="c", subcore_axis_name="s")
    kern = pl.kernel(
        _sc_body,
        out_type=jax.ShapeDtypeStruct((E * SROW,), jnp.int32),
        mesh=mesh,
        compiler_params=pltpu.CompilerParams(needs_layout_passes=False),
        scratch_types=[
            pltpu.VMEM((CH,), jnp.int32),
            pltpu.VMEM((CH,), jnp.int32),
            pltpu.VMEM((EPC * HB,), jnp.int32),
            pltpu.VMEM((EPT * HB,), jnp.int32),
            pltpu.VMEM((EPC * SROW,), jnp.int32),
            pltpu.VMEM_SHARED((NSUB, EPC * HB), jnp.int32),
            pltpu.VMEM_SHARED((EPC * NCORE * SROW // NCORE,), jnp.int32),
        ],
    )
    return kern(pbits_flat, experts_flat)


def _stage3_body(idx_ref, probs_ref, thr_ref, colsum_ref, zsum_ref,
                 oidx_ref, oprob_ref, tok_ref, lb_ref, z_ref):
    t = pl.program_id(0)
    topi = idx_ref[...]
    p = probs_ref[...]
    pb = lax.bitcast_convert_type(p, jnp.int32)
    row = lax.broadcasted_iota(jnp.int32, (TILE, K), 0)
    col = lax.broadcasted_iota(jnp.int32, (TILE, K), 1)
    gid = t * (TILE * K) + row * K + col
    inv = (SLOTS - 1) - gid
    thr_i = thr_ref[...]
    tpk = jnp.zeros((TILE, K), jnp.int32)
    tik = jnp.zeros((TILE, K), jnp.int32)
    for e in range(E):
        sel = topi == e
        tpk = jnp.where(sel, thr_i[e:e + 1, 3:4], tpk)
        tik = jnp.where(sel, thr_i[e:e + 1, 4:5], tik)
    keep = (pb > tpk) | ((pb == tpk) & (inv >= tik))
    mod_idx = jnp.where(keep, topi, -1)
    oidx_ref[...] = mod_idx
    oprob_ref[...] = jnp.where(keep, p, 0.0)

    @pl.when(t == 0)
    def _():
        tok_ref[...] = jnp.zeros_like(tok_ref)

    top1 = mod_idx[:, 0:1]
    iota_e = lax.broadcasted_iota(jnp.int32, (TILE, E), 1)
    tok_ref[...] += jnp.sum((top1 == iota_e).astype(jnp.float32), axis=0,
                            keepdims=True)

    @pl.when(t == NT - 1)
    def _():
        lb = jnp.sum(tok_ref[...] * colsum_ref[...])
        lb_ref[...] = (lb * (LBW / (N * E))).reshape(1, 1)
        z_ref[...] = (zsum_ref[0, 0] * (ZW / N)).reshape(1, 1)


def _stage3(topi, probs, thr, colsum, zsum):
    return pl.pallas_call(
        _stage3_body,
        grid=(NT,),
        in_specs=[
            pl.BlockSpec((TILE, K), lambda t: (t, 0)),
            pl.BlockSpec((TILE, K), lambda t: (t, 0)),
            pl.BlockSpec((E, SROW), lambda t: (0, 0)),
            pl.BlockSpec((1, E), lambda t: (0, 0)),
            pl.BlockSpec((1, 1), lambda t: (0, 0)),
        ],
        out_specs=[
            pl.BlockSpec((TILE, K), lambda t: (t, 0)),
            pl.BlockSpec((TILE, K), lambda t: (t, 0)),
            pl.BlockSpec((1, E), lambda t: (0, 0)),
            pl.BlockSpec((1, 1), lambda t: (0, 0)),
            pl.BlockSpec((1, 1), lambda t: (0, 0)),
        ],
        out_shape=[
            jax.ShapeDtypeStruct((N, K), jnp.int32),
            jax.ShapeDtypeStruct((N, K), jnp.float32),
            jax.ShapeDtypeStruct((1, E), jnp.float32),
            jax.ShapeDtypeStruct((1, 1), jnp.float32),
            jax.ShapeDtypeStruct((1, 1), jnp.float32),
        ],
    )(topi, probs, thr, colsum, zsum)


def kernel(x, W):
    topi, probs, colsum, zsum = _stage1(x, W)
    pbits = lax.bitcast_convert_type(probs, jnp.int32)
    thr = _sc_capacity(pbits.reshape(-1), topi.reshape(-1))
    thr = thr.reshape(E, SROW)
    mod_idx, fprobs, tok, lb, z = _stage3(topi, probs, thr, colsum, zsum)
    return (mod_idx, fprobs, lb.reshape(()), z.reshape(()), tok.reshape(E))


# speedup vs baseline: 139.8744x; 1.2044x over previous
import functools

import jax
import jax.numpy as jnp
from jax import lax
from jax.experimental import pallas as pl
from jax.experimental.pallas import tpu as pltpu
from jax.experimental.pallas import tpu_sc as plsc

N = 32768
D = 768
E = 64
K = 8
CAP = 512
SLOTS = N * K
LBW = 0.01
ZW = 0.001

TILE = 512
NT = N // TILE

NCORE = 2
NSUB = 16
CH = SLOTS // NSUB
EPC = E // NCORE
EPT = EPC // NSUB
HB = 1024
SENT = 0x7FFFFFFF
SROW = 16


def _stage1_body(x_ref, w_ref, idx_ref, probs_ref, colsum_ref, zsum_ref):
    t = pl.program_id(0)
    x = x_ref[...]
    w = w_ref[...]
    logits = lax.dot_general(x, w, (((1,), (1,)), ((), ())),
                             preferred_element_type=jnp.float32)
    rowmax = jnp.max(logits, axis=-1, keepdims=True)
    ex = jnp.exp(logits - rowmax)
    sumex = jnp.sum(ex, axis=-1, keepdims=True)

    @pl.when(t == 0)
    def _():
        colsum_ref[...] = jnp.zeros_like(colsum_ref)
        zsum_ref[...] = jnp.zeros_like(zsum_ref)

    colsum_ref[...] += jnp.sum(ex / sumex, axis=0, keepdims=True)
    lse = jnp.log(sumex) + rowmax
    zsum_ref[...] += jnp.sum(lse * lse).reshape(1, 1)

    iota = lax.broadcasted_iota(jnp.int32, (TILE, E), 1)
    cur = logits
    vals = []
    idxs = []
    for _k in range(K):
        m = jnp.max(cur, axis=-1, keepdims=True)
        am = jnp.min(jnp.where(cur == m, iota, E), axis=-1, keepdims=True)
        vals.append(m)
        idxs.append(am)
        cur = jnp.where(iota == am, -jnp.inf, cur)
    v = jnp.concatenate(vals, axis=1)
    topi = jnp.concatenate(idxs, axis=1)
    ev = jnp.exp(v - v[:, :1])
    p = ev / jnp.sum(ev, axis=-1, keepdims=True)
    p = p / jnp.maximum(jnp.sum(p, axis=-1, keepdims=True), 1e-8)
    idx_ref[...] = topi
    probs_ref[...] = p


def _stage1(x, W):
    return pl.pallas_call(
        _stage1_body,
        grid=(NT,),
        in_specs=[
            pl.BlockSpec((TILE, D), lambda t: (t, 0)),
            pl.BlockSpec((E, D), lambda t: (0, 0)),
        ],
        out_specs=[
            pl.BlockSpec((TILE, K), lambda t: (t, 0)),
            pl.BlockSpec((TILE, K), lambda t: (t, 0)),
            pl.BlockSpec((1, E), lambda t: (0, 0)),
            pl.BlockSpec((1, 1), lambda t: (0, 0)),
        ],
        out_shape=[
            jax.ShapeDtypeStruct((N, K), jnp.int32),
            jax.ShapeDtypeStruct((N, K), jnp.float32),
            jax.ShapeDtypeStruct((1, E), jnp.float32),
            jax.ShapeDtypeStruct((1, 1), jnp.float32),
        ],
    )(x, W)


def _lane_field(row, k):
    lane = lax.iota(jnp.int32, 16)
    return jnp.sum(jnp.where(lane == k, row, 0))


def _sc_body(pbits_hbm, experts_hbm, out_hbm, mod_hbm, fp_hbm,
             pchunk, echunk, hist, macc, state, tokcnt, shist, sstate):
    c = lax.axis_index("c")
    wid = lax.axis_index("s")
    lane = lax.iota(jnp.int32, 16)
    zero16 = jnp.zeros((16,), jnp.int32)

    pltpu.sync_copy(pbits_hbm.at[pl.ds(wid * CH, CH)], pchunk)
    pltpu.sync_copy(experts_hbm.at[pl.ds(wid * CH, CH)], echunk)

    init_row = jnp.where(lane == 2, jnp.int32(CAP), 0)
    for e in range(EPC):
        state[pl.ds(e * SROW, 16)] = init_row

    e0 = wid * EPT
    off = e0 * HB

    for lvl in range(5):
        @pl.loop(0, EPC * HB, step=16)
        def _(i):
            hist[pl.ds(i, 16)] = zero16

        @pl.loop(0, CH, step=16)
        def _(i):
            pb = pchunk[pl.ds(i, 16)]
            ev = echunk[pl.ds(i, 16)]
            eloc = lax.bitwise_and(ev, EPC - 1)
            mycore = lax.shift_right_logical(ev, 5) == c
            tw = plsc.load_gather(state, [eloc * SROW])
            if lvl == 0:
                pend = mycore
                bucket = lax.shift_right_logical(pb, 20)
            elif lvl == 1:
                pend = mycore & (lax.shift_right_logical(pb, 20)
                                 == lax.shift_right_logical(tw, 20))
                bucket = lax.bitwise_and(lax.shift_right_logical(pb, 10), HB - 1)
            elif lvl == 2:
                pend = mycore & (lax.shift_right_logical(pb, 10)
                                 == lax.shift_right_logical(tw, 10))
                bucket = lax.bitwise_and(pb, HB - 1)
            elif lvl == 3:
                inv = (SLOTS - 1) - (wid * CH + i + lane)
                pend = mycore & (pb == tw)
                bucket = lax.shift_right_logical(inv, 8)
            else:
                tiw = plsc.load_gather(state, [eloc * SROW + 1])
                inv = (SLOTS - 1) - (wid * CH + i + lane)
                pend = mycore & (pb == tw) & (
                    lax.shift_right_logical(inv, 8)
                    == lax.shift_right_logical(tiw, 8))
                bucket = lax.bitwise_and(inv, 255)
            addr = eloc * HB + bucket
            cnt, last = plsc.scan_count(addr, mask=pend)
            plsc.addupdate_scatter(hist, [addr], cnt, mask=last)

        pltpu.sync_copy(hist, shist.at[wid])
        plsc.subcore_barrier()
        for t2 in range(NSUB):
            pltpu.sync_copy(shist.at[t2, pl.ds(off, EPT * HB)],
                            hist.at[pl.ds(t2 * EPT * HB, EPT * HB)])

        @pl.loop(0, EPT * HB, step=16)
        def _(j):
            acc = hist[pl.ds(j, 16)]
            for t2 in range(1, NSUB):
                acc = acc + hist[pl.ds(t2 * EPT * HB + j, 16)]
            macc[pl.ds(j, 16)] = acc

        for ex in range(EPT):
            eloc = e0 + ex
            row = state[pl.ds(eloc * SROW, 16)]
            tpbw = _lane_field(row, 0)
            tinvw = _lane_field(row, 1)
            r = _lane_field(row, 2)
            tpbf = _lane_field(row, 3)
            tinvf = _lane_field(row, 4)
            notdone = tpbw != SENT

            def scan_step(jj, carry, _ex=ex, _r=r):
                suffix, found, bstar, sexcl = carry
                j = 63 - jj
                v = macc[pl.ds(_ex * HB + j * 16, 16)]
                vrev = lax.rev(v, (0,))
                cum = plsc.cumsum(vrev)
                tot = jnp.sum(v)
                hit = (found == 0) & (suffix + tot >= _r)
                target = _r - suffix
                ffs = jnp.max(plsc.all_reduce_ffs(cum >= target))
                before = jnp.sum(jnp.where(lane < ffs, vrev, 0))
                cand_b = j * 16 + (15 - ffs)
                cand_se = suffix + before
                bstar = jnp.where(hit, cand_b, bstar)
                sexcl = jnp.where(hit, cand_se, sexcl)
                found = found | jnp.where(hit, 1, 0)
                return suffix + tot, found, bstar, sexcl

            z = jnp.int32(0)
            _, found, bstar, sexcl = lax.fori_loop(0, 64, scan_step,
                                                   (z, z, z, z))
            keepall = found == 0
            newr = r - sexcl
            if lvl == 0:
                ntpbw = lax.shift_left(bstar, 20)
                ntinvw = tinvw
            elif lvl == 1:
                ntpbw = tpbw | lax.shift_left(bstar, 10)
                ntinvw = tinvw
            elif lvl == 2:
                ntpbw = tpbw | bstar
                ntinvw = tinvw
            elif lvl == 3:
                ntpbw = tpbw
                ntinvw = lax.shift_left(bstar, 8)
            else:
                ntpbw = tpbw
                ntinvw = tinvw | bstar
            if lvl <= 2:
                ntpbf, ntinvf = ntpbw, tinvf
            else:
                ntpbf, ntinvf = tpbf, ntinvw
            live = notdone & jnp.logical_not(keepall)
            wtpbw = jnp.where(notdone,
                              jnp.where(keepall, jnp.int32(SENT), ntpbw), tpbw)
            wtinvw = jnp.where(live, ntinvw, tinvw)
            wr = jnp.where(live, newr, r)
            wtpbf = jnp.where(live, ntpbf, tpbf)
            wtinvf = jnp.where(live, ntinvf, tinvf)
            nrow = jnp.where(
                lane == 0, wtpbw,
                jnp.where(lane == 1, wtinvw,
                jnp.where(lane == 2, wr,
                jnp.where(lane == 3, wtpbf,
                jnp.where(lane == 4, wtinvf,
                jnp.where(lane == 5, lax.shift_right_logical(wtpbf, 15),
                jnp.where(lane == 6, lax.bitwise_and(wtpbf, 0x7FFF), 0)))))))
            state[pl.ds(eloc * SROW, 16)] = nrow

        pltpu.sync_copy(state.at[pl.ds(e0 * SROW, EPT * SROW)],
                        sstate.at[pl.ds(e0 * SROW, EPT * SROW)])
        plsc.subcore_barrier()
        pltpu.sync_copy(sstate, state)

    for j in range(EPC // 16):
        tokcnt[pl.ds(j * 16, 16)] = zero16

    @pl.loop(0, CH, step=16)
    def _(i):
        pb = pchunk[pl.ds(i, 16)]
        ev = echunk[pl.ds(i, 16)]
        eloc = lax.bitwise_and(ev, EPC - 1)
        mycore = lax.shift_right_logical(ev, 5) == c
        tp = plsc.load_gather(state, [eloc * SROW + 3])
        ti = plsc.load_gather(state, [eloc * SROW + 4])
        gid = wid * CH + i + lane
        inv = (SLOTS - 1) - gid
        keep = mycore & ((pb > tp) | ((pb == tp) & (inv >= ti)))
        echunk[pl.ds(i, 16)] = jnp.where(keep, ev + 1, 0)
        pchunk[pl.ds(i, 16)] = jnp.where(keep, pb, 0)
        t1 = keep & (lax.bitwise_and(gid, K - 1) == 0)
        cnt1, last1 = plsc.scan_count(eloc, mask=t1)
        plsc.addupdate_scatter(tokcnt, [eloc], cnt1, mask=last1)

    pltpu.sync_copy(echunk, mod_hbm.at[c, pl.ds(wid * CH, CH)])
    pltpu.sync_copy(pchunk, fp_hbm.at[c, pl.ds(wid * CH, CH)])

    plsc.subcore_barrier()
    pltpu.sync_copy(tokcnt, sstate.at[pl.ds(wid * EPC, EPC)])
    plsc.subcore_barrier()
    pltpu.sync_copy(sstate, hist.at[pl.ds(0, NSUB * EPC)])
    for ex in range(EPT):
        eloc = e0 + ex
        pertile = plsc.load_gather(hist, [lane * EPC + eloc])
        total = jnp.sum(pertile)
        row = state[pl.ds(eloc * SROW, 16)]
        state[pl.ds(eloc * SROW, 16)] = jnp.where(lane == 7, total, row)

    pltpu.sync_copy(state.at[pl.ds(e0 * SROW, EPT * SROW)],
                    out_hbm.at[pl.ds((c * EPC + e0) * SROW, EPT * SROW)])


def _sc_capacity(pbits_flat, experts_flat):
    mesh = plsc.VectorSubcoreMesh(core_axis_name="c", subcore_axis_name="s")
    kern = pl.kernel(
        _sc_body,
        out_type=[
            jax.ShapeDtypeStruct((E * SROW,), jnp.int32),
            jax.ShapeDtypeStruct((NCORE, SLOTS), jnp.int32),
            jax.ShapeDtypeStruct((NCORE, SLOTS), jnp.int32),
        ],
        mesh=mesh,
        compiler_params=pltpu.CompilerParams(needs_layout_passes=False),
        scratch_types=[
            pltpu.VMEM((CH,), jnp.int32),
            pltpu.VMEM((CH,), jnp.int32),
            pltpu.VMEM((EPC * HB,), jnp.int32),
            pltpu.VMEM((EPT * HB,), jnp.int32),
            pltpu.VMEM((EPC * SROW,), jnp.int32),
            pltpu.VMEM((EPC,), jnp.int32),
            pltpu.VMEM_SHARED((NSUB, EPC * HB), jnp.int32),
            pltpu.VMEM_SHARED((NSUB * EPC,), jnp.int32),
        ],
    )
    return kern(pbits_flat, experts_flat)


def _losses_body(colsum_ref, zsum_ref, tok_ref, lb_ref, z_ref):
    lb = jnp.sum(colsum_ref[...] * tok_ref[...])
    lb_ref[...] = (lb * (LBW / (N * E))).reshape(1, 1)
    z_ref[...] = zsum_ref[...] * (ZW / N)


def _losses(colsum, zsum, tokf):
    return pl.pallas_call(
        _losses_body,
        out_shape=[
            jax.ShapeDtypeStruct((1, 1), jnp.float32),
            jax.ShapeDtypeStruct((1, 1), jnp.float32),
        ],
    )(colsum, zsum, tokf)


def kernel(x, W):
    topi, probs, colsum, zsum = _stage1(x, W)
    pbits = lax.bitcast_convert_type(probs, jnp.int32)
    thr, mod_enc, fp_enc = _sc_capacity(pbits.reshape(-1), topi.reshape(-1))
    thr = thr.reshape(E, SROW)
    mod_idx = ((mod_enc[0] | mod_enc[1]) - 1).reshape(N, K)
    fprobs = lax.bitcast_convert_type(fp_enc[0] | fp_enc[1],
                                      jnp.float32).reshape(N, K)
    tokf = thr[:, 7].astype(jnp.float32)
    lb, z = _losses(colsum, zsum, tokf.reshape(1, E))
    return (mod_idx, fprobs, lb.reshape(()), z.reshape(()), tokf)
